# baseline (speedup 1.0000x reference)
import jax
import jax.numpy as jnp
from jax import lax
from jax.experimental import pallas as pl
from jax.experimental.pallas import tpu as pltpu
from jax.experimental.pallas import tpu_sc as plsc

_B = 16384
_D = 128
_NC = 2
_NS = 16
_NW = _NC * _NS
_LANES = 16
_CHUNK = 128
_PER_W = _B // _NW
_NCHUNK = _PER_W // _CHUNK


def _sc_body(i_hbm, j_hbm, k_hbm, u_hbm, v_hbm, out_hbm,
             ib, jb, kb, u_buf0, u_buf1, vj_buf0, vj_buf1,
             vk_buf0, vk_buf1, out_buf, sem0, sem1):
    wid = lax.axis_index("s") * _NC + lax.axis_index("c")
    ci = pltpu.async_copy(i_hbm.at[wid], ib, sem0)
    cj = pltpu.async_copy(j_hbm.at[wid], jb, sem0)
    ck = pltpu.async_copy(k_hbm.at[wid], kb, sem0)
    ci.wait()
    cj.wait()
    ck.wait()

    ubufs = (u_buf0, u_buf1)
    jbufs = (vj_buf0, vj_buf1)
    kbufs = (vk_buf0, vk_buf1)
    sems = (sem0, sem1)
    lane = lax.iota(jnp.int32, _LANES)

    def start_chunk(c, q):
        pltpu.async_copy(u_hbm.at[ib.at[c]], ubufs[q], sems[q])
        pltpu.async_copy(v_hbm.at[jb.at[c]], jbufs[q], sems[q])
        pltpu.async_copy(v_hbm.at[kb.at[c]], kbufs[q], sems[q])

    def wait_chunk(c, q):
        pltpu.make_async_copy(u_hbm.at[ib.at[c]], ubufs[q], sems[q]).wait()
        pltpu.make_async_copy(v_hbm.at[jb.at[c]], jbufs[q], sems[q]).wait()
        pltpu.make_async_copy(v_hbm.at[kb.at[c]], kbufs[q], sems[q]).wait()

    def compute_chunk(c, q):
        ub, jbf, kbf = ubufs[q], jbufs[q], kbufs[q]

        def gbody(g, carry):
            rows = lane + g * _LANES
            def sblock(b, accs, rows=rows):
                accs = list(accs)
                for ss in range(32):
                    dcol = (lane + (b * 32 + ss)) & (_D - 1)
                    uc = plsc.load_gather(ub, [rows, dcol])
                    vjc = plsc.load_gather(jbf, [rows, dcol])
                    vkc = plsc.load_gather(kbf, [rows, dcol])
                    accs[ss % 4] = accs[ss % 4] + uc * (vjc - vkc)
                return tuple(accs)

            zero = jnp.zeros((_LANES,), jnp.float32)
            accs = lax.fori_loop(0, _D // 32, sblock, (zero, zero, zero, zero))
            acc = (accs[0] + accs[1]) + (accs[2] + accs[3])
            sig = 1.0 / (1.0 + jnp.exp(-acc))
            out_buf[pl.ds(g * _LANES, _LANES)] = sig
            return carry

        lax.fori_loop(0, _CHUNK // _LANES, gbody, 0)
        pltpu.sync_copy(out_buf, out_hbm.at[wid, c])

    start_chunk(0, 0)
    start_chunk(1, 1)

    def pbody(p, carry):
        for q in range(2):
            c = 2 * p + q
            wait_chunk(c, q)
            compute_chunk(c, q)

            @pl.when(c + 2 < _NCHUNK)
            def _(c=c, q=q):
                start_chunk(c + 2, q)
        return carry

    lax.fori_loop(0, _NCHUNK // 2, pbody, 0)


@jax.jit
def kernel(i, j, k, u_weight, v_weight):
    i3 = i.reshape(_NW, _NCHUNK, _CHUNK)
    j3 = j.reshape(_NW, _NCHUNK, _CHUNK)
    k3 = k.reshape(_NW, _NCHUNK, _CHUNK)
    run = pl.kernel(
        _sc_body,
        out_type=jax.ShapeDtypeStruct((_NW, _NCHUNK, _CHUNK), jnp.float32),
        mesh=plsc.VectorSubcoreMesh(core_axis_name="c", subcore_axis_name="s"),
        scratch_types=[
            pltpu.VMEM((_NCHUNK, _CHUNK), jnp.int32),
            pltpu.VMEM((_NCHUNK, _CHUNK), jnp.int32),
            pltpu.VMEM((_NCHUNK, _CHUNK), jnp.int32),
            pltpu.VMEM((_CHUNK, _D), jnp.float32),
            pltpu.VMEM((_CHUNK, _D), jnp.float32),
            pltpu.VMEM((_CHUNK, _D), jnp.float32),
            pltpu.VMEM((_CHUNK, _D), jnp.float32),
            pltpu.VMEM((_CHUNK, _D), jnp.float32),
            pltpu.VMEM((_CHUNK, _D), jnp.float32),
            pltpu.VMEM((_CHUNK,), jnp.float32),
            pltpu.SemaphoreType.DMA,
            pltpu.SemaphoreType.DMA,
        ],
        compiler_params=pltpu.CompilerParams(
            needs_layout_passes=False,
            disable_bounds_checks=True,
            disable_semaphore_checks=True,
            skip_device_barrier=True,
        ),
    )
    out = run(i3, j3, k3, u_weight, v_weight)
    return out.reshape(_B)


# speedup vs baseline: 1.0121x; 1.0121x over previous
import jax
import jax.numpy as jnp
from jax import lax
from jax.experimental import pallas as pl
from jax.experimental.pallas import tpu as pltpu
from jax.experimental.pallas import tpu_sc as plsc

_B = 16384
_D = 128
_NC = 2
_NS = 16
_NW = _NC * _NS
_LANES = 16
_CHUNK = 128
_PER_W = _B // _NW
_NCHUNK = _PER_W // _CHUNK


def _sc_body(i_hbm, j_hbm, k_hbm, u_hbm, v_hbm, out_hbm,
             ib, jb, kb, u2, vj2, vk2, out_buf, sems):
    wid = lax.axis_index("s") * _NC + lax.axis_index("c")
    ci = pltpu.async_copy(i_hbm.at[wid], ib, sems.at[0])
    cj = pltpu.async_copy(j_hbm.at[wid], jb, sems.at[0])
    ck = pltpu.async_copy(k_hbm.at[wid], kb, sems.at[0])
    ci.wait()
    cj.wait()
    ck.wait()

    lane = lax.iota(jnp.int32, _LANES)

    def start_chunk(c):
        q = c & 1
        pltpu.async_copy(u_hbm.at[ib.at[c]], u2.at[q], sems.at[q])
        pltpu.async_copy(v_hbm.at[jb.at[c]], vj2.at[q], sems.at[q])
        pltpu.async_copy(v_hbm.at[kb.at[c]], vk2.at[q], sems.at[q])

    def wait_chunk(c):
        q = c & 1
        pltpu.make_async_copy(u_hbm.at[ib.at[c]], u2.at[q], sems.at[q]).wait()
        pltpu.make_async_copy(v_hbm.at[jb.at[c]], vj2.at[q], sems.at[q]).wait()
        pltpu.make_async_copy(v_hbm.at[kb.at[c]], vk2.at[q], sems.at[q]).wait()

    def compute_chunk(c):
        qs = jnp.zeros((_LANES,), jnp.int32) + (c & 1)

        def gbody(g, carry):
            rows = lane + g * _LANES
            def sblock(b, accs, rows=rows):
                accs = list(accs)
                for ss in range(32):
                    dcol = (lane + (b * 32 + ss)) & (_D - 1)
                    uc = plsc.load_gather(u2, [qs, rows, dcol])
                    vjc = plsc.load_gather(vj2, [qs, rows, dcol])
                    vkc = plsc.load_gather(vk2, [qs, rows, dcol])
                    accs[ss % 4] = accs[ss % 4] + uc * (vjc - vkc)
                return tuple(accs)

            zero = jnp.zeros((_LANES,), jnp.float32)
            accs = lax.fori_loop(0, _D // 32, sblock, (zero, zero, zero, zero))
            acc = (accs[0] + accs[1]) + (accs[2] + accs[3])
            sig = 1.0 / (1.0 + jnp.exp(-acc))
            out_buf[pl.ds(g * _LANES, _LANES)] = sig
            return carry

        lax.fori_loop(0, _CHUNK // _LANES, gbody, 0)
        pltpu.sync_copy(out_buf, out_hbm.at[wid, c])

    start_chunk(0)
    start_chunk(1)

    def cbody(c, carry):
        wait_chunk(c)
        compute_chunk(c)

        @pl.when(c + 2 < _NCHUNK)
        def _():
            start_chunk(c + 2)

        return carry

    lax.fori_loop(0, _NCHUNK, cbody, 0)


@jax.jit
def kernel(i, j, k, u_weight, v_weight):
    i3 = i.reshape(_NW, _NCHUNK, _CHUNK)
    j3 = j.reshape(_NW, _NCHUNK, _CHUNK)
    k3 = k.reshape(_NW, _NCHUNK, _CHUNK)
    run = pl.kernel(
        _sc_body,
        out_type=jax.ShapeDtypeStruct((_NW, _NCHUNK, _CHUNK), jnp.float32),
        mesh=plsc.VectorSubcoreMesh(core_axis_name="c", subcore_axis_name="s"),
        scratch_types=[
            pltpu.VMEM((_NCHUNK, _CHUNK), jnp.int32),
            pltpu.VMEM((_NCHUNK, _CHUNK), jnp.int32),
            pltpu.VMEM((_NCHUNK, _CHUNK), jnp.int32),
            pltpu.VMEM((2, _CHUNK, _D), jnp.float32),
            pltpu.VMEM((2, _CHUNK, _D), jnp.float32),
            pltpu.VMEM((2, _CHUNK, _D), jnp.float32),
            pltpu.VMEM((_CHUNK,), jnp.float32),
            pltpu.SemaphoreType.DMA((2,)),
        ],
        compiler_params=pltpu.CompilerParams(needs_layout_passes=False),
    )
    out = run(i3, j3, k3, u_weight, v_weight)
    return out.reshape(_B)
